# Initial kernel scaffold; baseline (speedup 1.0000x reference)
import jax
import jax.numpy as jnp
from jax.experimental import pallas as pl


def kernel(preds, regs, img_dims):
    raise NotImplementedError("write your pallas kernel here")


# speedup vs baseline: 104.8628x; 104.8628x over previous
import numpy as np
import jax
import jax.numpy as jnp
from jax.experimental import pallas as pl
from jax.experimental.pallas import tpu as pltpu

_STRIDE = 16
_FH, _FW, _NA = 37, 62, 9
_N = _FH * _FW * _NA
_K = 6000
_BLK = 512
_NB = 12
_KPAD = _BLK * _NB
_ROWS = _KPAD // 128
_RPB = _BLK // 128
_OUTP = 384
_KEEP_POST = 300
_IOU_T = 0.7


def _np_default_boxes():
    a = []
    for r in (0.5, 1.0, 2.0):
        for s in (0.5, 1.0, 2.0):
            w = _STRIDE * s * np.sqrt(1.0 / r)
            h = _STRIDE * s * np.sqrt(r)
            a.append([-w / 2.0, -h / 2.0, w / 2.0, h / 2.0])
    anchors = np.array(a, dtype=np.float32)
    sx = (np.arange(_FW) * _STRIDE).astype(np.float32)
    sy = (np.arange(_FH) * _STRIDE).astype(np.float32)
    gx, gy = np.meshgrid(sx, sy)
    shifts = np.stack([gx, gy, gx, gy], axis=-1)
    boxes = shifts[:, :, None, :] + anchors[None, None, :, :]
    return boxes.reshape(-1, 4).astype(np.float32)


_DBOXES = _np_default_boxes()

_DN = (((1,), (0,)), ((), ()))


def _det_body(wh_ref, logit_ref, dbox_ref, reg_ref, out_ref, alive_ref,
              x1_ref, y1_ref, x2_ref, y2_ref, ar_ref, rank_ref, sel_ref):
    Hc = wh_ref[0]
    Wc = wh_ref[1]
    db0 = dbox_ref[0, 0]
    db1 = dbox_ref[0, 1]
    db2 = dbox_ref[0, 2]
    db3 = dbox_ref[0, 3]
    rg0 = reg_ref[0, 0]
    rg1 = reg_ref[0, 1]
    rg2 = reg_ref[0, 2]
    rg3 = reg_ref[0, 3]

    w = db2 - db0
    h = db3 - db1
    cx = db0 + 0.5 * w
    cy = db1 + 0.5 * h
    pcx = rg0 * w + cx
    pcy = rg1 * h + cy
    pw = jnp.exp(rg2) * w
    ph = jnp.exp(rg3) * h
    x1 = jnp.clip(pcx - 0.5 * pw, 0.0, Wc)
    y1 = jnp.clip(pcy - 0.5 * ph, 0.0, Hc)
    x2 = jnp.clip(pcx + 0.5 * pw, 0.0, Wc)
    y2 = jnp.clip(pcy + 0.5 * ph, 0.0, Hc)
    area = (x2 - x1) * (y2 - y1)
    x1_ref[...] = x1
    y1_ref[...] = y1
    x2_ref[...] = x2
    y2_ref[...] = y2
    ar_ref[...] = area

    lane = jax.lax.broadcasted_iota(jnp.int32, (_ROWS, 128), 1)
    rowi = jax.lax.broadcasted_iota(jnp.int32, (_ROWS, 128), 0)
    jidx = rowi * 128 + lane
    alive_ref[...] = (jidx < _K).astype(jnp.float32)

    a_io = jax.lax.broadcasted_iota(jnp.int32, (_BLK, _BLK), 0)
    b_io = jax.lax.broadcasted_iota(jnp.int32, (_BLK, _BLK), 1)
    tri = (a_io < b_io).astype(jnp.float32)

    def blk(ref, i):
        return ref[pl.ds(i * _RPB, _RPB), :].reshape(_BLK)

    def rowbc(v):
        return jax.lax.broadcast_in_dim(v, (_BLK, _BLK), (1,))

    def colmats(i):
        return tuple(jnp.transpose(rowbc(blk(r, i)))
                     for r in (x1_ref, y1_ref, x2_ref, y2_ref, ar_ref))

    def sup_matrix(cm, j):
        xa1, ya1, xa2, ya2, aa = cm
        xb1 = rowbc(blk(x1_ref, j))
        yb1 = rowbc(blk(y1_ref, j))
        xb2 = rowbc(blk(x2_ref, j))
        yb2 = rowbc(blk(y2_ref, j))
        ab = rowbc(blk(ar_ref, j))
        iw = jnp.maximum(jnp.minimum(xa2, xb2) - jnp.maximum(xa1, xb1), 0.0)
        ih = jnp.maximum(jnp.minimum(ya2, yb2) - jnp.maximum(ya1, yb1), 0.0)
        inter = iw * ih
        iou = inter / (aa + ab - inter + 1e-9)
        return (iou > _IOU_T).astype(jnp.float32)

    def outer(i, carry):
        init = alive_ref[pl.ds(i * _RPB, _RPB), :].reshape(_BLK)
        cm = colmats(i)
        S = sup_matrix(cm, i) * tri

        def w_cond(c):
            return c[0]

        def w_body(c):
            _, cur = c
            cnt = jax.lax.dot_general(cur.reshape(1, _BLK), S, _DN,
                                      preferred_element_type=jnp.float32)
            new = init * (cnt.reshape(_BLK) == 0.0).astype(jnp.float32)
            changed = jnp.sum(jnp.abs(new - cur)) > 0.0
            return (changed, new)

        _, fin = jax.lax.while_loop(w_cond, w_body, (True, init))
        alive_ref[pl.ds(i * _RPB, _RPB), :] = fin.reshape(_RPB, 128)
        arow = fin.reshape(1, _BLK)

        def inner(j, _):
            vb = alive_ref[pl.ds(j * _RPB, _RPB), :].reshape(_BLK)
            Sc = sup_matrix(cm, j)
            cnt = jax.lax.dot_general(arow, Sc, _DN,
                                      preferred_element_type=jnp.float32).reshape(_BLK)
            nvb = vb * (cnt == 0.0).astype(jnp.float32)
            alive_ref[pl.ds(j * _RPB, _RPB), :] = nvb.reshape(_RPB, 128)
            return 0

        jax.lax.fori_loop(i + 1, _NB, inner, 0)
        return carry

    jax.lax.fori_loop(0, _NB, outer, 0)

    alive = alive_ref[...]
    mrows = (jax.lax.broadcasted_iota(jnp.int32, (_ROWS, _ROWS), 1) <
             jax.lax.broadcasted_iota(jnp.int32, (_ROWS, _ROWS), 0)).astype(jnp.float32)
    lstrict = (jax.lax.broadcasted_iota(jnp.int32, (128, 128), 0) <
               jax.lax.broadcasted_iota(jnp.int32, (128, 128), 1)).astype(jnp.float32)
    ones128 = jnp.ones((128, 128), jnp.float32)
    prev_rows = jax.lax.dot_general(mrows, alive, _DN, preferred_element_type=jnp.float32)
    rank_ref[...] = (
        jax.lax.dot_general(prev_rows, ones128, _DN, preferred_element_type=jnp.float32)
        + jax.lax.dot_general(alive, lstrict, _DN, preferred_element_type=jnp.float32))
    conf = (logit_ref[0] > 0.0).astype(jnp.float32)
    sel_ref[...] = alive * conf

    r_io = jax.lax.broadcasted_iota(jnp.int32, (_OUTP, _BLK), 0).astype(jnp.float32)

    def obc(v):
        return jax.lax.broadcast_in_dim(v, (_OUTP, _BLK), (1,))

    for c, cref in enumerate((x1_ref, y1_ref, x2_ref, y2_ref)):
        def body(jb, acc, cref=cref):
            P = (obc(blk(rank_ref, jb)) == r_io).astype(jnp.float32) * obc(blk(sel_ref, jb))
            contrib = jnp.sum(P * obc(blk(cref, jb)), axis=1)
            return acc + contrib

        acc = jax.lax.fori_loop(0, _NB, body, jnp.zeros((_OUTP,), jnp.float32))
        out_ref[0, c, :] = acc


@jax.jit
def _run(wh, logitp, dboxp, regp):
    bs = logitp.shape[0]
    return pl.pallas_call(
        _det_body,
        grid=(bs,),
        in_specs=[
            pl.BlockSpec(memory_space=pltpu.SMEM),
            pl.BlockSpec((1, _ROWS, 128), lambda i: (i, 0, 0)),
            pl.BlockSpec((1, 4, _ROWS, 128), lambda i: (i, 0, 0, 0)),
            pl.BlockSpec((1, 4, _ROWS, 128), lambda i: (i, 0, 0, 0)),
        ],
        out_specs=pl.BlockSpec((1, 4, _OUTP), lambda i: (i, 0, 0)),
        out_shape=jax.ShapeDtypeStruct((bs, 4, _OUTP), jnp.float32),
        scratch_shapes=[pltpu.VMEM((_ROWS, 128), jnp.float32)
                        for _ in range(8)],
    )(wh, logitp, dboxp, regp)


def kernel(preds, regs, img_dims):
    bs = preds.shape[0]
    logits = preds.reshape(bs, _N)
    vals, idx = jax.lax.top_k(logits, _K)
    pad = _KPAD - _K
    svals = jnp.concatenate(
        [vals, jnp.full((bs, pad), -jnp.inf, vals.dtype)], axis=1)
    sidx = jnp.concatenate([idx, jnp.zeros((bs, pad), idx.dtype)], axis=1)
    rflat = regs.reshape(bs, _N, 4)
    sregs = jnp.take_along_axis(rflat, sidx[..., None], axis=1)
    sdbox = jnp.take(jnp.asarray(_DBOXES), sidx, axis=0)
    regp = sregs.transpose(0, 2, 1).reshape(bs, 4, _ROWS, 128)
    dboxp = sdbox.transpose(0, 2, 1).reshape(bs, 4, _ROWS, 128)
    logitp = svals.reshape(bs, _ROWS, 128)
    wh = jnp.stack([img_dims[0], img_dims[1]]).astype(jnp.float32)
    out = _run(wh, logitp, dboxp, regp)
    return out.transpose(0, 2, 1)[:, :_KEEP_POST, :]


# speedup vs baseline: 181.6453x; 1.7322x over previous
import numpy as np
import jax
import jax.numpy as jnp
from jax.experimental import pallas as pl
from jax.experimental.pallas import tpu as pltpu

_STRIDE = 16
_FH, _FW, _NA = 37, 62, 9
_N = _FH * _FW * _NA
_K = 6000
_BLK = 512
_NB = 12
_KPAD = _BLK * _NB
_OUTP = 384
_KEEP_POST = 300
_IOU_T = 0.7


def _np_default_boxes():
    a = []
    for r in (0.5, 1.0, 2.0):
        for s in (0.5, 1.0, 2.0):
            w = _STRIDE * s * np.sqrt(1.0 / r)
            h = _STRIDE * s * np.sqrt(r)
            a.append([-w / 2.0, -h / 2.0, w / 2.0, h / 2.0])
    anchors = np.array(a, dtype=np.float32)
    sx = (np.arange(_FW) * _STRIDE).astype(np.float32)
    sy = (np.arange(_FH) * _STRIDE).astype(np.float32)
    gx, gy = np.meshgrid(sx, sy)
    shifts = np.stack([gx, gy, gx, gy], axis=-1)
    boxes = shifts[:, :, None, :] + anchors[None, None, :, :]
    return boxes.reshape(-1, 4).astype(np.float32)


_DBOXES = _np_default_boxes()

_DN = (((1,), (0,)), ((), ()))


def _det_body(wh_ref, logit_ref, dbox_ref, reg_ref, out_ref, alive_ref,
              x1_ref, y1_ref, x2_ref, y2_ref, ar_ref, rank_ref, sel_ref):
    Hc = wh_ref[0]
    Wc = wh_ref[1]
    db0 = dbox_ref[0, 0]
    db1 = dbox_ref[0, 1]
    db2 = dbox_ref[0, 2]
    db3 = dbox_ref[0, 3]
    rg0 = reg_ref[0, 0]
    rg1 = reg_ref[0, 1]
    rg2 = reg_ref[0, 2]
    rg3 = reg_ref[0, 3]

    w = db2 - db0
    h = db3 - db1
    cx = db0 + 0.5 * w
    cy = db1 + 0.5 * h
    pcx = rg0 * w + cx
    pcy = rg1 * h + cy
    pw = jnp.exp(rg2) * w
    ph = jnp.exp(rg3) * h
    x1 = jnp.clip(pcx - 0.5 * pw, 0.0, Wc)
    y1 = jnp.clip(pcy - 0.5 * ph, 0.0, Hc)
    x2 = jnp.clip(pcx + 0.5 * pw, 0.0, Wc)
    y2 = jnp.clip(pcy + 0.5 * ph, 0.0, Hc)
    area = (x2 - x1) * (y2 - y1)
    x1_ref[...] = x1
    y1_ref[...] = y1
    x2_ref[...] = x2
    y2_ref[...] = y2
    ar_ref[...] = area

    lane = jax.lax.broadcasted_iota(jnp.int32, (_NB, _BLK), 1)
    rowi = jax.lax.broadcasted_iota(jnp.int32, (_NB, _BLK), 0)
    jidx = rowi * _BLK + lane
    alive_ref[...] = (jidx < _K).astype(jnp.float32)

    a_io = jax.lax.broadcasted_iota(jnp.int32, (_BLK, _BLK), 0)
    b_io = jax.lax.broadcasted_iota(jnp.int32, (_BLK, _BLK), 1)
    tri = (a_io < b_io).astype(jnp.float32)

    def blk(ref, i):
        return ref[pl.ds(i, 1), :]

    def rowbc(v):
        return jnp.broadcast_to(v, (_BLK, _BLK))

    def colmats(i):
        return tuple(jnp.transpose(rowbc(blk(r, i)))
                     for r in (x1_ref, y1_ref, x2_ref, y2_ref, ar_ref))

    def sup_matrix(cm, j):
        xa1, ya1, xa2, ya2, aa = cm
        xb1 = rowbc(blk(x1_ref, j))
        yb1 = rowbc(blk(y1_ref, j))
        xb2 = rowbc(blk(x2_ref, j))
        yb2 = rowbc(blk(y2_ref, j))
        ab = rowbc(blk(ar_ref, j))
        iw = jnp.maximum(jnp.minimum(xa2, xb2) - jnp.maximum(xa1, xb1), 0.0)
        ih = jnp.maximum(jnp.minimum(ya2, yb2) - jnp.maximum(ya1, yb1), 0.0)
        inter = iw * ih
        iou = inter / (aa + ab - inter + 1e-9)
        return (iou > _IOU_T).astype(jnp.float32)

    def o_cond(c):
        i, kept = c
        return (i < _NB) & (kept < float(_KEEP_POST))

    def o_body(c):
        i, kept = c
        init = blk(alive_ref, i)
        cm = colmats(i)
        S = sup_matrix(cm, i) * tri

        def w_cond(wc):
            return wc[0]

        def w_body(wc):
            _, cur = wc
            cnt = jax.lax.dot_general(cur, S, _DN,
                                      preferred_element_type=jnp.float32)
            new = init * (cnt == 0.0).astype(jnp.float32)
            changed = jnp.sum(jnp.abs(new - cur)) > 0.0
            return (changed, new)

        _, fin = jax.lax.while_loop(w_cond, w_body, (True, init))
        alive_ref[pl.ds(i, 1), :] = fin

        def inner(j, _):
            vb = blk(alive_ref, j)
            Sc = sup_matrix(cm, j)
            cnt = jax.lax.dot_general(fin, Sc, _DN,
                                      preferred_element_type=jnp.float32)
            alive_ref[pl.ds(j, 1), :] = vb * (cnt == 0.0).astype(jnp.float32)
            return 0

        jax.lax.fori_loop(i + 1, _NB, inner, 0)
        return (i + 1, kept + jnp.sum(fin))

    jax.lax.while_loop(o_cond, o_body, (0, 0.0))

    alive = alive_ref[...]
    mrows = (jax.lax.broadcasted_iota(jnp.int32, (_NB, _NB), 1) <
             jax.lax.broadcasted_iota(jnp.int32, (_NB, _NB), 0)).astype(jnp.float32)
    lstrict = tri
    ones_b = jnp.ones((_BLK, _BLK), jnp.float32)
    prev_rows = jax.lax.dot_general(mrows, alive, _DN, preferred_element_type=jnp.float32)
    rank_ref[...] = (
        jax.lax.dot_general(prev_rows, ones_b, _DN, preferred_element_type=jnp.float32)
        + jax.lax.dot_general(alive, lstrict, _DN, preferred_element_type=jnp.float32))
    conf = (logit_ref[0] > 0.0).astype(jnp.float32)
    sel_ref[...] = alive * conf

    r_io = jax.lax.broadcasted_iota(jnp.int32, (_OUTP, _BLK), 0).astype(jnp.float32)

    def obc(v):
        return jnp.broadcast_to(v, (_OUTP, _BLK))

    def out_body(jb, accs):
        P = (obc(blk(rank_ref, jb)) == r_io).astype(jnp.float32) * obc(blk(sel_ref, jb))
        a0, a1, a2, a3 = accs
        return (a0 + jnp.sum(P * obc(blk(x1_ref, jb)), axis=1),
                a1 + jnp.sum(P * obc(blk(y1_ref, jb)), axis=1),
                a2 + jnp.sum(P * obc(blk(x2_ref, jb)), axis=1),
                a3 + jnp.sum(P * obc(blk(y2_ref, jb)), axis=1))

    z = jnp.zeros((_OUTP,), jnp.float32)
    accs = jax.lax.fori_loop(0, _NB, out_body, (z, z, z, z))
    for c in range(4):
        out_ref[0, c, :] = accs[c]


@jax.jit
def _run(wh, logitp, dboxp, regp):
    bs = logitp.shape[0]
    return pl.pallas_call(
        _det_body,
        grid=(bs,),
        in_specs=[
            pl.BlockSpec(memory_space=pltpu.SMEM),
            pl.BlockSpec((1, _NB, _BLK), lambda i: (i, 0, 0)),
            pl.BlockSpec((1, 4, _NB, _BLK), lambda i: (i, 0, 0, 0)),
            pl.BlockSpec((1, 4, _NB, _BLK), lambda i: (i, 0, 0, 0)),
        ],
        out_specs=pl.BlockSpec((1, 4, _OUTP), lambda i: (i, 0, 0)),
        out_shape=jax.ShapeDtypeStruct((bs, 4, _OUTP), jnp.float32),
        scratch_shapes=[pltpu.VMEM((_NB, _BLK), jnp.float32)
                        for _ in range(8)],
    )(wh, logitp, dboxp, regp)


def kernel(preds, regs, img_dims):
    bs = preds.shape[0]
    logits = preds.reshape(bs, _N)
    vals, idx = jax.lax.top_k(logits, _K)
    pad = _KPAD - _K
    svals = jnp.concatenate(
        [vals, jnp.full((bs, pad), -jnp.inf, vals.dtype)], axis=1)
    sidx = jnp.concatenate([idx, jnp.zeros((bs, pad), idx.dtype)], axis=1)
    rflat = regs.reshape(bs, _N, 4)
    sregs = jnp.take_along_axis(rflat, sidx[..., None], axis=1)
    sdbox = jnp.take(jnp.asarray(_DBOXES), sidx, axis=0)
    regp = sregs.transpose(0, 2, 1).reshape(bs, 4, _NB, _BLK)
    dboxp = sdbox.transpose(0, 2, 1).reshape(bs, 4, _NB, _BLK)
    logitp = svals.reshape(bs, _NB, _BLK)
    wh = jnp.stack([img_dims[0], img_dims[1]]).astype(jnp.float32)
    out = _run(wh, logitp, dboxp, regp)
    return out.transpose(0, 2, 1)[:, :_KEEP_POST, :]


# speedup vs baseline: 253.1587x; 1.3937x over previous
import numpy as np
import jax
import jax.numpy as jnp
from jax.experimental import pallas as pl
from jax.experimental.pallas import tpu as pltpu

_STRIDE = 16
_FH, _FW, _NA = 37, 62, 9
_N = _FH * _FW * _NA
_K = 6000
_BLK = 512
_NB = 12
_KPAD = _BLK * _NB
_OUTP = 384
_KEEP_POST = 300
_IOU_T = 0.7


_DN = (((1,), (0,)), ((), ()))


def _det_body(wh_ref, logit_ref, sidx_ref, reg_ref, out_ref, alive_ref,
              x1_ref, y1_ref, x2_ref, y2_ref, ar_ref, rank_ref, sel_ref):
    Hc = wh_ref[0]
    Wc = wh_ref[1]
    sidx = sidx_ref[0]
    rg0 = reg_ref[0, 0]
    rg1 = reg_ref[0, 1]
    rg2 = reg_ref[0, 2]
    rg3 = reg_ref[0, 3]

    a_id = sidx % _NA
    grid = sidx // _NA
    col = grid % _FW
    rowg = grid // _FW
    gx = (col * _STRIDE).astype(jnp.float32)
    gy = (rowg * _STRIDE).astype(jnp.float32)
    ax1 = jnp.zeros(sidx.shape, jnp.float32)
    ay1 = jnp.zeros(sidx.shape, jnp.float32)
    for a in range(_NA):
        r = (0.5, 1.0, 2.0)[a // 3]
        s = (0.5, 1.0, 2.0)[a % 3]
        aw2 = np.float32(-(_STRIDE * s * np.sqrt(1.0 / r)) / 2.0)
        ah2 = np.float32(-(_STRIDE * s * np.sqrt(r)) / 2.0)
        m = a_id == a
        ax1 = jnp.where(m, aw2, ax1)
        ay1 = jnp.where(m, ah2, ay1)
    dx1 = gx + ax1
    dy1 = gy + ay1
    dx2 = gx - ax1
    dy2 = gy - ay1
    w = dx2 - dx1
    h = dy2 - dy1
    cx = dx1 + 0.5 * w
    cy = dy1 + 0.5 * h
    pcx = rg0 * w + cx
    pcy = rg1 * h + cy
    pw = jnp.exp(rg2) * w
    ph = jnp.exp(rg3) * h
    x1 = jnp.clip(pcx - 0.5 * pw, 0.0, Wc)
    y1 = jnp.clip(pcy - 0.5 * ph, 0.0, Hc)
    x2 = jnp.clip(pcx + 0.5 * pw, 0.0, Wc)
    y2 = jnp.clip(pcy + 0.5 * ph, 0.0, Hc)
    area = (x2 - x1) * (y2 - y1)
    x1_ref[...] = x1
    y1_ref[...] = y1
    x2_ref[...] = x2
    y2_ref[...] = y2
    ar_ref[...] = area

    lane = jax.lax.broadcasted_iota(jnp.int32, (_NB, _BLK), 1)
    rowi = jax.lax.broadcasted_iota(jnp.int32, (_NB, _BLK), 0)
    jidx = rowi * _BLK + lane
    alive_ref[...] = (jidx < _K).astype(jnp.float32)

    a_io = jax.lax.broadcasted_iota(jnp.int32, (_BLK, _BLK), 0)
    b_io = jax.lax.broadcasted_iota(jnp.int32, (_BLK, _BLK), 1)
    tri = (a_io < b_io).astype(jnp.float32)

    def blk(ref, i):
        return ref[pl.ds(i, 1), :]

    def rowbc(v):
        return jnp.broadcast_to(v, (_BLK, _BLK))

    def colmats(i):
        return tuple(jnp.transpose(rowbc(blk(r, i)))
                     for r in (x1_ref, y1_ref, x2_ref, y2_ref, ar_ref))

    def sup_matrix(cm, j):
        xa1, ya1, xa2, ya2, aa = cm
        xb1 = rowbc(blk(x1_ref, j))
        yb1 = rowbc(blk(y1_ref, j))
        xb2 = rowbc(blk(x2_ref, j))
        yb2 = rowbc(blk(y2_ref, j))
        ab = rowbc(blk(ar_ref, j))
        iw = jnp.maximum(jnp.minimum(xa2, xb2) - jnp.maximum(xa1, xb1), 0.0)
        ih = jnp.maximum(jnp.minimum(ya2, yb2) - jnp.maximum(ya1, yb1), 0.0)
        inter = iw * ih
        iou = inter / (aa + ab - inter + 1e-9)
        return (iou > _IOU_T).astype(jnp.float32)

    def o_cond(c):
        i, kept = c
        return (i < _NB) & (kept < float(_KEEP_POST))

    def o_body(c):
        i, kept = c
        init = blk(alive_ref, i)
        cm = colmats(i)
        S = sup_matrix(cm, i) * tri

        def w_cond(wc):
            return wc[0]

        def w_body(wc):
            _, cur = wc
            cnt = jax.lax.dot_general(cur, S, _DN,
                                      preferred_element_type=jnp.float32)
            new = init * (cnt == 0.0).astype(jnp.float32)
            changed = jnp.sum(jnp.abs(new - cur)) > 0.0
            return (changed, new)

        _, fin = jax.lax.while_loop(w_cond, w_body, (True, init))
        alive_ref[pl.ds(i, 1), :] = fin

        def inner(j, _):
            vb = blk(alive_ref, j)
            Sc = sup_matrix(cm, j)
            cnt = jax.lax.dot_general(fin, Sc, _DN,
                                      preferred_element_type=jnp.float32)
            alive_ref[pl.ds(j, 1), :] = vb * (cnt == 0.0).astype(jnp.float32)
            return 0

        jax.lax.fori_loop(i + 1, _NB, inner, 0)
        return (i + 1, kept + jnp.sum(fin))

    jax.lax.while_loop(o_cond, o_body, (0, 0.0))

    alive = alive_ref[...]
    mrows = (jax.lax.broadcasted_iota(jnp.int32, (_NB, _NB), 1) <
             jax.lax.broadcasted_iota(jnp.int32, (_NB, _NB), 0)).astype(jnp.float32)
    lstrict = tri
    ones_b = jnp.ones((_BLK, _BLK), jnp.float32)
    prev_rows = jax.lax.dot_general(mrows, alive, _DN, preferred_element_type=jnp.float32)
    rank_ref[...] = (
        jax.lax.dot_general(prev_rows, ones_b, _DN, preferred_element_type=jnp.float32)
        + jax.lax.dot_general(alive, lstrict, _DN, preferred_element_type=jnp.float32))
    conf = (logit_ref[0] > 0.0).astype(jnp.float32)
    sel_ref[...] = alive * conf

    r_io = jax.lax.broadcasted_iota(jnp.int32, (_OUTP, _BLK), 0).astype(jnp.float32)

    def obc(v):
        return jnp.broadcast_to(v, (_OUTP, _BLK))

    def out_body(jb, accs):
        P = (obc(blk(rank_ref, jb)) == r_io).astype(jnp.float32) * obc(blk(sel_ref, jb))
        a0, a1, a2, a3 = accs
        return (a0 + jnp.sum(P * obc(blk(x1_ref, jb)), axis=1),
                a1 + jnp.sum(P * obc(blk(y1_ref, jb)), axis=1),
                a2 + jnp.sum(P * obc(blk(x2_ref, jb)), axis=1),
                a3 + jnp.sum(P * obc(blk(y2_ref, jb)), axis=1))

    z = jnp.zeros((_OUTP,), jnp.float32)
    accs = jax.lax.fori_loop(0, _NB, out_body, (z, z, z, z))
    for c in range(4):
        out_ref[0, c, :] = accs[c]


@jax.jit
def _run(wh, logitp, sidxp, regp):
    bs = logitp.shape[0]
    return pl.pallas_call(
        _det_body,
        grid=(bs,),
        in_specs=[
            pl.BlockSpec(memory_space=pltpu.SMEM),
            pl.BlockSpec((1, _NB, _BLK), lambda i: (i, 0, 0)),
            pl.BlockSpec((1, _NB, _BLK), lambda i: (i, 0, 0)),
            pl.BlockSpec((1, 4, _NB, _BLK), lambda i: (i, 0, 0, 0)),
        ],
        out_specs=pl.BlockSpec((1, 4, _OUTP), lambda i: (i, 0, 0)),
        out_shape=jax.ShapeDtypeStruct((bs, 4, _OUTP), jnp.float32),
        scratch_shapes=[pltpu.VMEM((_NB, _BLK), jnp.float32)
                        for _ in range(8)],
    )(wh, logitp, sidxp, regp)


def kernel(preds, regs, img_dims):
    bs = preds.shape[0]
    logits = preds.reshape(bs, _N)
    vals, idx = jax.lax.top_k(logits, _K)
    pad = _KPAD - _K
    svals = jnp.concatenate(
        [vals, jnp.full((bs, pad), -jnp.inf, vals.dtype)], axis=1)
    sidx = jnp.concatenate([idx, jnp.zeros((bs, pad), idx.dtype)], axis=1)
    rflat = regs.reshape(bs, _N, 4)
    sregs = jnp.take_along_axis(rflat, sidx[..., None], axis=1)
    regp = sregs.transpose(0, 2, 1).reshape(bs, 4, _NB, _BLK)
    sidxp = sidx.astype(jnp.int32).reshape(bs, _NB, _BLK)
    logitp = svals.reshape(bs, _NB, _BLK)
    wh = jnp.stack([img_dims[0], img_dims[1]]).astype(jnp.float32)
    out = _run(wh, logitp, sidxp, regp)
    return out.transpose(0, 2, 1)[:, :_KEEP_POST, :]


# speedup vs baseline: 280.0416x; 1.1062x over previous
import numpy as np
import jax
import jax.numpy as jnp
from jax.experimental import pallas as pl
from jax.experimental.pallas import tpu as pltpu

_STRIDE = 16
_FH, _FW, _NA = 37, 62, 9
_N = _FH * _FW * _NA
_K = 6000
_BLK = 512
_NB = 12
_KPAD = _BLK * _NB
_OUTP = 384
_KEEP_POST = 300
_IOU_T = 0.7


_DN = (((1,), (0,)), ((), ()))


def _det_body(wh_ref, logit_ref, sidx_ref, reg_ref, out_ref, alive_ref,
              x1_ref, y1_ref, x2_ref, y2_ref, ar_ref, rank_ref, sel_ref):
    Hc = wh_ref[0]
    Wc = wh_ref[1]
    sidx = sidx_ref[0]
    rg0 = reg_ref[0, 0]
    rg1 = reg_ref[0, 1]
    rg2 = reg_ref[0, 2]
    rg3 = reg_ref[0, 3]

    a_id = sidx % _NA
    grid = sidx // _NA
    col = grid % _FW
    rowg = grid // _FW
    gx = (col * _STRIDE).astype(jnp.float32)
    gy = (rowg * _STRIDE).astype(jnp.float32)
    ax1 = jnp.zeros(sidx.shape, jnp.float32)
    ay1 = jnp.zeros(sidx.shape, jnp.float32)
    for a in range(_NA):
        r = (0.5, 1.0, 2.0)[a // 3]
        s = (0.5, 1.0, 2.0)[a % 3]
        aw2 = np.float32(-(_STRIDE * s * np.sqrt(1.0 / r)) / 2.0)
        ah2 = np.float32(-(_STRIDE * s * np.sqrt(r)) / 2.0)
        m = a_id == a
        ax1 = jnp.where(m, aw2, ax1)
        ay1 = jnp.where(m, ah2, ay1)
    dx1 = gx + ax1
    dy1 = gy + ay1
    dx2 = gx - ax1
    dy2 = gy - ay1
    w = dx2 - dx1
    h = dy2 - dy1
    cx = dx1 + 0.5 * w
    cy = dy1 + 0.5 * h
    pcx = rg0 * w + cx
    pcy = rg1 * h + cy
    pw = jnp.exp(rg2) * w
    ph = jnp.exp(rg3) * h
    x1 = jnp.clip(pcx - 0.5 * pw, 0.0, Wc)
    y1 = jnp.clip(pcy - 0.5 * ph, 0.0, Hc)
    x2 = jnp.clip(pcx + 0.5 * pw, 0.0, Wc)
    y2 = jnp.clip(pcy + 0.5 * ph, 0.0, Hc)
    area = (x2 - x1) * (y2 - y1)
    x1_ref[...] = x1
    y1_ref[...] = y1
    x2_ref[...] = x2
    y2_ref[...] = y2
    ar_ref[...] = area

    lane = jax.lax.broadcasted_iota(jnp.int32, (_NB, _BLK), 1)
    rowi = jax.lax.broadcasted_iota(jnp.int32, (_NB, _BLK), 0)
    jidx = rowi * _BLK + lane
    alive_ref[...] = (jidx < _K).astype(jnp.float32)

    a_io = jax.lax.broadcasted_iota(jnp.int32, (_BLK, _BLK), 0)
    b_io = jax.lax.broadcasted_iota(jnp.int32, (_BLK, _BLK), 1)
    tri = (a_io < b_io).astype(jnp.float32)

    def blk(ref, i):
        return ref[pl.ds(i, 1), :]

    def rowbc(v):
        return jnp.broadcast_to(v, (_BLK, _BLK))

    def colmats(i):
        return tuple(jnp.transpose(rowbc(blk(r, i)))
                     for r in (x1_ref, y1_ref, x2_ref, y2_ref, ar_ref))

    def sup_matrix(cm, j):
        xa1, ya1, xa2, ya2, aa = cm
        xb1 = rowbc(blk(x1_ref, j))
        yb1 = rowbc(blk(y1_ref, j))
        xb2 = rowbc(blk(x2_ref, j))
        yb2 = rowbc(blk(y2_ref, j))
        ab = rowbc(blk(ar_ref, j))
        iw = jnp.maximum(jnp.minimum(xa2, xb2) - jnp.maximum(xa1, xb1), 0.0)
        ih = jnp.maximum(jnp.minimum(ya2, yb2) - jnp.maximum(ya1, yb1), 0.0)
        inter = iw * ih
        iou = inter / (aa + ab - inter + 1e-9)
        return (iou > _IOU_T).astype(jnp.float32)

    def o_cond(c):
        i, kept = c
        return (i < _NB) & (kept < float(_KEEP_POST))

    def o_body(c):
        i, kept = c
        init = blk(alive_ref, i)
        cm = colmats(i)
        S = sup_matrix(cm, i) * tri

        def w_cond(wc):
            return wc[0]

        def w_body(wc):
            _, cur = wc
            cnt = jax.lax.dot_general(cur, S, _DN,
                                      preferred_element_type=jnp.float32)
            new = init * (cnt == 0.0).astype(jnp.float32)
            changed = jnp.sum(jnp.abs(new - cur)) > 0.0
            return (changed, new)

        _, fin = jax.lax.while_loop(w_cond, w_body, (True, init))
        alive_ref[pl.ds(i, 1), :] = fin

        def inner(j, _):
            vb = blk(alive_ref, j)
            Sc = sup_matrix(cm, j)
            cnt = jax.lax.dot_general(fin, Sc, _DN,
                                      preferred_element_type=jnp.float32)
            alive_ref[pl.ds(j, 1), :] = vb * (cnt == 0.0).astype(jnp.float32)
            return 0

        jax.lax.fori_loop(i + 1, _NB, inner, 0)
        return (i + 1, kept + jnp.sum(fin))

    n_proc, _ = jax.lax.while_loop(o_cond, o_body, (0, 0.0))

    alive = alive_ref[...]
    mrows = (jax.lax.broadcasted_iota(jnp.int32, (_NB, _NB), 1) <
             jax.lax.broadcasted_iota(jnp.int32, (_NB, _NB), 0)).astype(jnp.float32)
    lstrict = tri
    ones_b = jnp.ones((_BLK, _BLK), jnp.float32)
    prev_rows = jax.lax.dot_general(mrows, alive, _DN, preferred_element_type=jnp.float32)
    rank_ref[...] = (
        jax.lax.dot_general(prev_rows, ones_b, _DN, preferred_element_type=jnp.float32)
        + jax.lax.dot_general(alive, lstrict, _DN, preferred_element_type=jnp.float32))
    conf = (logit_ref[0] > 0.0).astype(jnp.float32)
    sel_ref[...] = alive * conf

    r_io = jax.lax.broadcasted_iota(jnp.int32, (_OUTP, _BLK), 0).astype(jnp.float32)

    def obc(v):
        return jnp.broadcast_to(v, (_OUTP, _BLK))

    def out_body(jb, accs):
        P = (obc(blk(rank_ref, jb)) == r_io).astype(jnp.float32) * obc(blk(sel_ref, jb))
        a0, a1, a2, a3 = accs
        return (a0 + jnp.sum(P * obc(blk(x1_ref, jb)), axis=1),
                a1 + jnp.sum(P * obc(blk(y1_ref, jb)), axis=1),
                a2 + jnp.sum(P * obc(blk(x2_ref, jb)), axis=1),
                a3 + jnp.sum(P * obc(blk(y2_ref, jb)), axis=1))

    z = jnp.zeros((_OUTP,), jnp.float32)
    accs = jax.lax.fori_loop(0, n_proc, out_body, (z, z, z, z))
    for c in range(4):
        out_ref[0, c, :] = accs[c]


@jax.jit
def _run(wh, logitp, sidxp, regp):
    bs = logitp.shape[0]
    return pl.pallas_call(
        _det_body,
        grid=(bs,),
        in_specs=[
            pl.BlockSpec(memory_space=pltpu.SMEM),
            pl.BlockSpec((1, _NB, _BLK), lambda i: (i, 0, 0)),
            pl.BlockSpec((1, _NB, _BLK), lambda i: (i, 0, 0)),
            pl.BlockSpec((1, 4, _NB, _BLK), lambda i: (i, 0, 0, 0)),
        ],
        out_specs=pl.BlockSpec((1, 4, _OUTP), lambda i: (i, 0, 0)),
        out_shape=jax.ShapeDtypeStruct((bs, 4, _OUTP), jnp.float32),
        scratch_shapes=[pltpu.VMEM((_NB, _BLK), jnp.float32)
                        for _ in range(8)],
    )(wh, logitp, sidxp, regp)


def kernel(preds, regs, img_dims):
    bs = preds.shape[0]
    logits = preds.reshape(bs, _N)
    vals, idx = jax.lax.top_k(logits, _K)
    pad = _KPAD - _K
    svals = jnp.concatenate(
        [vals, jnp.full((bs, pad), -jnp.inf, vals.dtype)], axis=1)
    sidx = jnp.concatenate([idx, jnp.zeros((bs, pad), idx.dtype)], axis=1)
    rflat = regs.reshape(bs, _N, 4)
    sregs = jnp.take_along_axis(rflat, sidx[..., None], axis=1)
    regp = sregs.transpose(0, 2, 1).reshape(bs, 4, _NB, _BLK)
    sidxp = sidx.astype(jnp.int32).reshape(bs, _NB, _BLK)
    logitp = svals.reshape(bs, _NB, _BLK)
    wh = jnp.stack([img_dims[0], img_dims[1]]).astype(jnp.float32)
    out = _run(wh, logitp, sidxp, regp)
    return out.transpose(0, 2, 1)[:, :_KEEP_POST, :]
